# Initial kernel scaffold; baseline (speedup 1.0000x reference)
import jax
import jax.numpy as jnp
from jax.experimental import pallas as pl


def kernel(x, edge_index, W1, b1, W_mu, b_mu, W_sig, b_sig):
    raise NotImplementedError("write your pallas kernel here")


# speedup vs baseline: 2.6681x; 2.6681x over previous
import functools

import jax
import jax.numpy as jnp
from jax.experimental import pallas as pl
from jax.experimental.pallas import tpu as pltpu

N_BLOCK = 1000


def _mm_kernel(x_ref, w_ref, o_ref):
    o_ref[...] = jnp.dot(x_ref[...], w_ref[...],
                         preferred_element_type=jnp.float32)


def _matmul(x, w):
    n, k = x.shape
    m = w.shape[1]
    grid = (n // N_BLOCK,)
    return pl.pallas_call(
        _mm_kernel,
        grid=grid,
        in_specs=[
            pl.BlockSpec((N_BLOCK, k), lambda i: (i, 0)),
            pl.BlockSpec((k, m), lambda i: (0, 0)),
        ],
        out_specs=pl.BlockSpec((N_BLOCK, m), lambda i: (i, 0)),
        out_shape=jax.ShapeDtypeStruct((n, m), jnp.float32),
    )(x, w)


def kernel(x, edge_index, W1, b1, W_mu, b_mu, W_sig, b_sig):
    n = x.shape[0]
    src = edge_index[0].astype(jnp.int32)
    dst = edge_index[1].astype(jnp.int32)

    ones = jnp.ones(src.shape[0], dtype=jnp.float32)
    deg = jax.ops.segment_sum(ones, dst, num_segments=n) + 1.0
    dinv = jax.lax.rsqrt(deg)

    def gcn(h, W, b):
        u = _matmul(h, W) * dinv[:, None]
        agg = jax.ops.segment_sum(u[src], dst, num_segments=n) + u
        return agg * dinv[:, None] + b

    h = jax.nn.relu(gcn(x, W1, b1))
    W2 = jnp.concatenate([W_mu, W_sig], axis=1)
    b2 = jnp.concatenate([b_mu, b_sig], axis=0)
    out = gcn(h, W2, b2)
    return (out[:, :128], out[:, 128:])


# speedup vs baseline: 10.6476x; 3.9906x over previous
import functools

import jax
import jax.numpy as jnp
from jax import lax
from jax.experimental import pallas as pl
from jax.experimental.pallas import tpu as pltpu
from jax.experimental.pallas import tpu_sc as plsc

_N = 10000
_E = 160000
_NC = 2
_NS = 16
_L = 16
_CH = 128
_EPT = _E // _NS
_CHUNK = 128
_NFULL = _EPT // _CHUNK
_TAIL = _EPT - _NFULL * _CHUNK
_NP = 10240
_RPT = _NP // _NS
_NB = 1000


def _sc_mesh():
    return plsc.VectorSubcoreMesh(core_axis_name="c", subcore_axis_name="s")


_EPW = _E // (_NC * _NS)
_DNF = _EPW // _CHUNK
_DTAIL = _EPW - _DNF * _CHUNK


def _sc_degree(dst, ones_hbm, zeros_hbm):
    @functools.partial(
        pl.kernel,
        out_type=jax.ShapeDtypeStruct((_NC, _NP, _CH), jnp.float32),
        mesh=_sc_mesh(),
        scratch_types=[
            pltpu.VMEM((_CHUNK,), jnp.int32),
            pltpu.VMEM((_DTAIL,), jnp.int32),
            pltpu.VMEM((_CHUNK, _CH), jnp.float32),
            pltpu.VMEM_SHARED((_NP, _CH), jnp.float32),
        ],
    )
    def body(dst_hbm, ones_hbm_ref, zeros_hbm_ref, out_hbm, dstv, dstv_t,
             onesv, acc):
        cid = lax.axis_index("c")
        sid = lax.axis_index("s")

        pltpu.sync_copy(ones_hbm_ref, onesv)
        pltpu.sync_copy(zeros_hbm_ref.at[pl.ds(sid * _RPT, _RPT), :],
                        acc.at[pl.ds(sid * _RPT, _RPT), :])
        plsc.subcore_barrier()

        ebase = (cid * _NS + sid) * _EPW

        def step(k, c):
            off = ebase + k * _CHUNK
            pltpu.sync_copy(dst_hbm.at[pl.ds(off, _CHUNK)], dstv)
            pltpu.sync_copy(onesv, acc.at[dstv], add=True)
            return c

        lax.fori_loop(0, _DNF, step, 0)
        toff = ebase + _DNF * _CHUNK
        pltpu.sync_copy(dst_hbm.at[pl.ds(toff, _DTAIL)], dstv_t)
        pltpu.sync_copy(onesv.at[pl.ds(0, _DTAIL), :], acc.at[dstv_t],
                        add=True)
        plsc.subcore_barrier()
        pltpu.sync_copy(acc.at[pl.ds(sid * _RPT, _RPT), :],
                        out_hbm.at[cid, pl.ds(sid * _RPT, _RPT), :])

    return body(dst, ones_hbm, zeros_hbm)


def _sc_aggregate(u_flat, src2, dst):
    @functools.partial(
        pl.kernel,
        out_type=jax.ShapeDtypeStruct((_NC, _NP, _CH), jnp.float32),
        mesh=_sc_mesh(),
        scratch_types=[
            pltpu.VMEM((_CHUNK,), jnp.int32),
            pltpu.VMEM((_CHUNK,), jnp.int32),
            pltpu.VMEM((_TAIL,), jnp.int32),
            pltpu.VMEM((_TAIL,), jnp.int32),
            pltpu.VMEM((_CHUNK, _CH), jnp.float32),
            pltpu.VMEM((_TAIL, _CH), jnp.float32),
            pltpu.VMEM_SHARED((_NP, _CH), jnp.float32),
            pltpu.SemaphoreType.DMA,
        ],
    )
    def body(u_hbm, src_hbm, dst_hbm, out_hbm, srcv, dstv, srcv_t, dstv_t,
             rows, rows_t, acc, sem):
        cid = lax.axis_index("c")
        sid = lax.axis_index("s")
        pltpu.sync_copy(u_hbm.at[pl.ds(cid * _NP + sid * _RPT, _RPT), :],
                        acc.at[pl.ds(sid * _RPT, _RPT), :])
        plsc.subcore_barrier()

        ebase = sid * _EPT
        sbase = cid * _E + ebase

        def step(k, c):
            off = k * _CHUNK
            pltpu.sync_copy(src_hbm.at[pl.ds(sbase + off, _CHUNK)], srcv)
            pltpu.sync_copy(dst_hbm.at[pl.ds(ebase + off, _CHUNK)], dstv)
            pltpu.async_copy(u_hbm.at[srcv], rows, sem).wait()
            pltpu.sync_copy(rows, acc.at[dstv], add=True)
            return c

        lax.fori_loop(0, _NFULL, step, 0)
        toff = _NFULL * _CHUNK
        pltpu.sync_copy(src_hbm.at[pl.ds(sbase + toff, _TAIL)], srcv_t)
        pltpu.sync_copy(dst_hbm.at[pl.ds(ebase + toff, _TAIL)], dstv_t)
        pltpu.async_copy(u_hbm.at[srcv_t], rows_t, sem).wait()
        pltpu.sync_copy(rows_t, acc.at[dstv_t], add=True)

        plsc.subcore_barrier()
        pltpu.sync_copy(acc.at[pl.ds(sid * _RPT, _RPT), :],
                        out_hbm.at[cid, pl.ds(sid * _RPT, _RPT), :])

    return body(u_flat, src2, dst)


def _dinv_of(deg_blk):
    return jax.lax.rsqrt(deg_blk[0, :, 0:1] + deg_blk[1, :, 0:1] + 1.0)


def _mm_scale_kernel(x_ref, w_ref, deg_ref, o_ref):
    t = jnp.dot(x_ref[...], w_ref[...], preferred_element_type=jnp.float32)
    u = t * _dinv_of(deg_ref)
    o_ref[0] = u[:, :_CH]
    o_ref[1] = u[:, _CH:]


def _mm_scale(x, w, deg16):
    return pl.pallas_call(
        _mm_scale_kernel,
        grid=(_N // _NB,),
        in_specs=[
            pl.BlockSpec((_NB, 256), lambda i: (i, 0)),
            pl.BlockSpec((256, 256), lambda i: (0, 0)),
            pl.BlockSpec((_NC, _NB, _CH), lambda i: (0, i, 0)),
        ],
        out_specs=pl.BlockSpec((_NC, _NB, _CH), lambda i: (0, i, 0)),
        out_shape=jax.ShapeDtypeStruct((_NC, _NP, _CH), jnp.float32),
    )(x, w, deg16)


def _mid_kernel(agg_ref, deg_ref, b1_ref, w_ref, o_ref):
    dinv = _dinv_of(deg_ref)
    h = jnp.concatenate([agg_ref[0], agg_ref[1]], axis=1)
    h = jnp.maximum(h * dinv + b1_ref[...], 0.0)
    t = jnp.dot(h, w_ref[...], preferred_element_type=jnp.float32)
    u = t * dinv
    o_ref[0] = u[:, :_CH]
    o_ref[1] = u[:, _CH:]


def _mid(agg1, deg16, b1, w2):
    return pl.pallas_call(
        _mid_kernel,
        grid=(_N // _NB,),
        in_specs=[
            pl.BlockSpec((_NC, _NB, _CH), lambda i: (0, i, 0)),
            pl.BlockSpec((_NC, _NB, _CH), lambda i: (0, i, 0)),
            pl.BlockSpec((1, 256), lambda i: (0, 0)),
            pl.BlockSpec((256, 256), lambda i: (0, 0)),
        ],
        out_specs=pl.BlockSpec((_NC, _NB, _CH), lambda i: (0, i, 0)),
        out_shape=jax.ShapeDtypeStruct((_NC, _NP, _CH), jnp.float32),
    )(agg1, deg16, b1, w2)


def _final_kernel(agg_ref, deg_ref, bmu_ref, bsig_ref, mu_ref, sig_ref):
    dinv = _dinv_of(deg_ref)
    mu_ref[...] = agg_ref[0] * dinv + bmu_ref[...]
    sig_ref[...] = agg_ref[1] * dinv + bsig_ref[...]


def _final(agg2, deg16, b_mu, b_sig):
    return pl.pallas_call(
        _final_kernel,
        grid=(_N // _NB,),
        in_specs=[
            pl.BlockSpec((_NC, _NB, _CH), lambda i: (0, i, 0)),
            pl.BlockSpec((_NC, _NB, _CH), lambda i: (0, i, 0)),
            pl.BlockSpec((1, _CH), lambda i: (0, 0)),
            pl.BlockSpec((1, _CH), lambda i: (0, 0)),
        ],
        out_specs=[
            pl.BlockSpec((_NB, _CH), lambda i: (i, 0)),
            pl.BlockSpec((_NB, _CH), lambda i: (i, 0)),
        ],
        out_shape=[
            jax.ShapeDtypeStruct((_N, _CH), jnp.float32),
            jax.ShapeDtypeStruct((_N, _CH), jnp.float32),
        ],
    )(agg2, deg16, b_mu, b_sig)


def kernel(x, edge_index, W1, b1, W_mu, b_mu, W_sig, b_sig):
    src = edge_index[0].astype(jnp.int32)
    dst = edge_index[1].astype(jnp.int32)
    src2 = jnp.concatenate([src, src + _NP])

    deg2 = _sc_degree(dst, jnp.ones((_CHUNK, _CH), jnp.float32),
                      jnp.zeros((_NP, _CH), jnp.float32))
    W2 = jnp.concatenate([W_mu, W_sig], axis=1)

    u1 = _sc_aggregate(_mm_scale(x, W1, deg2).reshape(_NC * _NP, _CH),
                       src2, dst)
    u2 = _sc_aggregate(_mid(u1, deg2, b1.reshape(1, -1), W2)
                       .reshape(_NC * _NP, _CH), src2, dst)
    return _final(u2, deg2, b_mu.reshape(1, _CH), b_sig.reshape(1, _CH))
